# Initial kernel scaffold; baseline (speedup 1.0000x reference)
import jax
import jax.numpy as jnp
from jax.experimental import pallas as pl


def kernel(node_feature, hop_feature, nn_idx_f2v, nn_idx_v2f, efeature_f2v, efeature_v2f, params):
    raise NotImplementedError("write your pallas kernel here")


# speedup vs baseline: 1.0758x; 1.0758x over previous
import functools

import jax
import jax.numpy as jnp
from jax import lax
from jax.experimental import pallas as pl
from jax.experimental.pallas import tpu as pltpu
from jax.experimental.pallas import tpu_sc as plsc

F32 = jnp.float32
NFEAT = 8
HOP = 6
NEDGE = 4
CHECK_LEN = 972
CODE_LEN = 1944
DV = 3
NTILES = 32


@functools.lru_cache(maxsize=1)
def _sc_mesh():
    return plsc.VectorSubcoreMesh(core_axis_name="c", subcore_axis_name="s")


def _tc_tables(x, wt):
    B, N, C = x.shape
    E, _, F = wt.shape
    TW = max(F, 128)

    def body(x_ref, w_ref, o_ref):
        y = jnp.dot(x_ref[0], w_ref[0], preferred_element_type=F32)
        if TW > F:
            y = jnp.concatenate([y, jnp.zeros((N, TW - F), F32)], axis=1)
        o_ref[0, 0] = y

    return pl.pallas_call(
        body,
        grid=(B, E),
        in_specs=[pl.BlockSpec((1, N, C), lambda b, e: (b, 0, 0)),
                  pl.BlockSpec((1, C, F), lambda b, e: (e, 0, 0))],
        out_specs=pl.BlockSpec((1, 1, N, TW), lambda b, e: (b, e, 0, 0)),
        out_shape=jax.ShapeDtypeStruct((B, E, N, TW), F32),
    )(x, wt)


def _tc_emodel(x, w1t, b1, w2t, b2):
    B, M, Cin = x.shape
    H = w1t.shape[1]
    E = w2t.shape[1]

    def body(x_ref, w1_ref, b1_ref, w2_ref, b2_ref, o_ref):
        h = jnp.maximum(
            jnp.dot(x_ref[0], w1_ref[...], preferred_element_type=F32)
            + b1_ref[...], 0.0)
        o_ref[0] = jnp.dot(h, w2_ref[...], preferred_element_type=F32) + b2_ref[...]

    return pl.pallas_call(
        body,
        grid=(B,),
        in_specs=[pl.BlockSpec((1, M, Cin), lambda b: (b, 0, 0)),
                  pl.BlockSpec((Cin, H), lambda b: (0, 0)),
                  pl.BlockSpec((1, H), lambda b: (0, 0)),
                  pl.BlockSpec((H, E), lambda b: (0, 0)),
                  pl.BlockSpec((1, E), lambda b: (0, 0))],
        out_specs=pl.BlockSpec((1, M, E), lambda b: (b, 0, 0)),
        out_shape=jax.ShapeDtypeStruct((B, M, E), F32),
    )(x, w1t, b1, w2t, b2)


def _tc_dense(x_v, x_f0, x_f1, vf0t, bf0, wv1t, vf1t, bf1, wf1t, uvt, bv):
    B, N, C = x_v.shape
    Nf = x_f0.shape[1]
    Cf = x_f0.shape[2]
    C1 = x_f1.shape[2]
    F = bv.shape[1]

    def body(xv_ref, xf0_ref, xf1_ref, vf0_ref, bf0_ref, wv1_ref, vf1_ref,
             bf1_ref, wf1_ref, uv_ref, bv_ref, s0_ref, addv_ref, f1_ref):
        xv = xv_ref[0]
        g1 = jnp.dot(xv, wv1_ref[...], preferred_element_type=F32)
        m1 = jnp.max(g1, axis=0, keepdims=True)
        s1 = jnp.dot(xf1_ref[0], vf1_ref[...], preferred_element_type=F32)
        f1 = jnp.maximum(m1 + s1 + bf1_ref[...], 0.0)
        f1_ref[0] = f1
        t1 = jnp.dot(f1, wf1_ref[...], preferred_element_type=F32)
        s0_ref[0] = (jnp.dot(xf0_ref[0], vf0_ref[...], preferred_element_type=F32)
                     + bf0_ref[...])
        addv_ref[0] = (jnp.dot(xv, uv_ref[...], preferred_element_type=F32)
                       + t1 + bv_ref[...])

    return pl.pallas_call(
        body,
        grid=(B,),
        in_specs=[pl.BlockSpec((1, N, C), lambda b: (b, 0, 0)),
                  pl.BlockSpec((1, Nf, Cf), lambda b: (b, 0, 0)),
                  pl.BlockSpec((1, 1, C1), lambda b: (b, 0, 0)),
                  pl.BlockSpec((Cf, F), lambda b: (0, 0)),
                  pl.BlockSpec((1, F), lambda b: (0, 0)),
                  pl.BlockSpec((C, F), lambda b: (0, 0)),
                  pl.BlockSpec((C1, F), lambda b: (0, 0)),
                  pl.BlockSpec((1, F), lambda b: (0, 0)),
                  pl.BlockSpec((F, F), lambda b: (0, 0)),
                  pl.BlockSpec((C, F), lambda b: (0, 0)),
                  pl.BlockSpec((1, F), lambda b: (0, 0))],
        out_specs=[pl.BlockSpec((1, Nf, F), lambda b: (b, 0, 0)),
                   pl.BlockSpec((1, N, F), lambda b: (b, 0, 0)),
                   pl.BlockSpec((1, 1, F), lambda b: (b, 0, 0))],
        out_shape=[jax.ShapeDtypeStruct((B, Nf, F), F32),
                   jax.ShapeDtypeStruct((B, N, F), F32),
                   jax.ShapeDtypeStruct((B, 1, F), F32)],
    )(x_v, x_f0, x_f1, vf0t, bf0, wv1t, vf1t, bf1, wf1t, uvt, bv)


def _tc_head(x_v, wt, node0b):
    B, N, F = x_v.shape

    def body(x_ref, w_ref, n_ref, o_ref):
        xs = x_ref[0, :CHECK_LEN, :]
        r = lax.dot_general(w_ref[...], xs, (((1,), (1,)), ((), ())),
                            preferred_element_type=F32)
        o_ref[0] = r + n_ref[0]

    return pl.pallas_call(
        body,
        grid=(B,),
        in_specs=[pl.BlockSpec((1, N, F), lambda b: (b, 0, 0)),
                  pl.BlockSpec((1, F), lambda b: (0, 0)),
                  pl.BlockSpec((1, 1, CHECK_LEN), lambda b: (b, 0, 0))],
        out_specs=pl.BlockSpec((1, 1, CHECK_LEN), lambda b: (b, 0, 0)),
        out_shape=jax.ShapeDtypeStruct((B, 1, CHECK_LEN), F32),
    )(x_v, wt, node0b)


def _sc_agg(table, idxm, etm, bias, F, K, NITER, SUB, PT):
    E = NEDGE
    RPD = SUB * K * E
    NTOT = NTILES * PT
    TW = max(F, 128)

    @functools.partial(
        pl.kernel,
        mesh=_sc_mesh(),
        out_type=jax.ShapeDtypeStruct((NTOT, F), F32),
        scratch_types=[
            pltpu.VMEM((NITER, RPD), jnp.int32),
            pltpu.VMEM((NITER * RPD + 16,), F32),
            pltpu.VMEM((PT, F), F32),
            pltpu.VMEM((PT, F), F32),
            pltpu.VMEM((RPD, TW), F32),
            pltpu.SemaphoreType.DMA,
        ],
    )
    def k(table_h, idx_h, et_h, bias_h, out_h, idx_v, et_v, bias_v, out_v,
          rows_v, sem):
        wid = lax.axis_index("s") * 2 + lax.axis_index("c")
        pltpu.sync_copy(idx_h.at[wid], idx_v)
        pltpu.sync_copy(et_h.at[wid], et_v)
        pltpu.sync_copy(bias_h.at[pl.ds(wid * PT, PT)], bias_v)

        ke = K * E

        def body(i, carry):
            pltpu.async_copy(table_h.at[idx_v.at[i]], rows_v, sem).wait()
            for d in range(SUB):
                base = d * ke
                off = i * RPD + base
                ch0 = et_v[pl.ds(off, 16)]
                ch1 = et_v[pl.ds(off + 8, 16)] if ke > 16 else None
                ets = [ch0[m] if m < 16 else ch1[m - 8] for m in range(ke)]
                dd = i * SUB + d
                for j in range(F // 16):
                    sl = pl.ds(j * 16, 16)
                    acc = None
                    for kk in range(K):
                        t = None
                        for e in range(E):
                            r = rows_v[base + kk * E + e, sl]
                            w = ets[kk * E + e]
                            t = r * w if t is None else t + r * w
                        acc = t if acc is None else jnp.maximum(acc, t)
                    out_v[dd, sl] = jnp.maximum(acc + bias_v[dd, sl], 0.0)
            return carry

        lax.fori_loop(0, NITER, body, 0)
        pltpu.sync_copy(out_v, out_h.at[pl.ds(wid * PT, PT)])

    return k(table, idxm, etm, bias)


def _pad_rows(x, n):
    return jnp.pad(x, ((0, n - x.shape[0]), (0, 0)))


def _edge_plan(ndst_total, k):
    sub = max(1, 48 // (k * NEDGE))
    pt = -(-ndst_total // (NTILES * sub)) * sub
    pt = -(-pt // 8) * 8
    return sub, pt, pt // sub


def kernel(node_feature, hop_feature, nn_idx_f2v, nn_idx_v2f, efeature_f2v,
           efeature_v2f, params):
    B = node_feature.shape[0]

    nf0 = node_feature[..., 0]
    x_v = jnp.transpose(nf0, (0, 2, 1))
    x_f0 = jnp.transpose(hop_feature[..., 0], (0, 2, 1))
    nhop = node_feature.reshape(B, CODE_LEN, NFEAT)
    x_f1 = nhop.mean(-1).reshape(B, 1, CODE_LEN)

    ef = jnp.transpose(efeature_f2v, (0, 2, 3, 1)).reshape(B, CODE_LEN * DV, HOP + 1)
    ev = jnp.transpose(efeature_v2f, (0, 2, 3, 1)).reshape(B, CHECK_LEN * HOP, HOP + 1)
    pf, pv = params['emodel_f2v'], params['emodel_v2f']
    et_f2v = _tc_emodel(ef, pf['w1'].T, pf['b1'].reshape(1, -1),
                        pf['w2'].T, pf['b2'].reshape(1, -1))
    et_v2f = _tc_emodel(ev, pv['w1'].T, pv['b1'].reshape(1, -1),
                        pv['w2'].T, pv['b2'].reshape(1, -1))

    sub_v, pt_v, ni_v = _edge_plan(B * CHECK_LEN, HOP)
    sub_f, pt_f, ni_f = _edge_plan(B * CODE_LEN, DV)
    rpd_v = sub_v * HOP * NEDGE
    rpd_f = sub_f * DV * NEDGE

    b_ar = jnp.arange(B, dtype=jnp.int32)[:, None, None, None]
    e_ar = jnp.arange(NEDGE, dtype=jnp.int32)[None, None, None, :]
    rows_v2f = ((b_ar * NEDGE + e_ar) * CODE_LEN
                + nn_idx_v2f.astype(jnp.int32)[..., None])
    idxm_v2f = _pad_rows(rows_v2f.reshape(B * CHECK_LEN, HOP * NEDGE),
                         NTILES * pt_v).reshape(NTILES, ni_v, rpd_v)
    rows_f2v = ((b_ar * NEDGE + e_ar) * CHECK_LEN
                + nn_idx_f2v.astype(jnp.int32)[..., None])
    idxm_f2v = _pad_rows(rows_f2v.reshape(B * CODE_LEN, DV * NEDGE),
                         NTILES * pt_f).reshape(NTILES, ni_f, rpd_f)
    etm_v2f = jnp.pad(
        _pad_rows(et_v2f.reshape(B * CHECK_LEN, HOP * NEDGE),
                  NTILES * pt_v).reshape(NTILES, ni_v * rpd_v),
        ((0, 0), (0, 16)))
    etm_f2v = jnp.pad(
        _pad_rows(et_f2v.reshape(B * CODE_LEN, DV * NEDGE),
                  NTILES * pt_f).reshape(NTILES, ni_f * rpd_f),
        ((0, 0), (0, 16)))

    for L in params['layers']:
        F = L['bv'].shape[0]
        s0b, addv, f1 = _tc_dense(
            x_v, x_f0, x_f1,
            L['Vf0'].T, L['bf0'].reshape(1, F),
            L['Wv2f1'][0].T, L['Vf1'].T, L['bf1'].reshape(1, F),
            L['Wf2v1'][0].T, L['Uv'].T, L['bv'].reshape(1, F))

        hv = _tc_tables(x_v, jnp.transpose(L['Wv2f0'], (0, 2, 1)))
        table_v = hv.reshape(B * NEDGE * CODE_LEN, hv.shape[-1])
        bias_v = _pad_rows(s0b.reshape(B * CHECK_LEN, F), NTILES * pt_v)
        f0_flat = _sc_agg(table_v, idxm_v2f, etm_v2f, bias_v,
                          F, HOP, ni_v, sub_v, pt_v)
        f0 = f0_flat[:B * CHECK_LEN].reshape(B, CHECK_LEN, F)

        hf = _tc_tables(f0, jnp.transpose(L['Wf2v0'], (0, 2, 1)))
        table_f = hf.reshape(B * NEDGE * CHECK_LEN, hf.shape[-1])
        bias_f = _pad_rows(addv.reshape(B * CODE_LEN, F), NTILES * pt_f)
        xv_flat = _sc_agg(table_f, idxm_f2v, etm_f2v, bias_f,
                          F, DV, ni_f, sub_f, pt_f)
        x_v = xv_flat[:B * CODE_LEN].reshape(B, CODE_LEN, F)
        x_f0, x_f1 = f0, f1

    node0b = (nf0[:, :1, :CHECK_LEN] + params['out_b'][0])
    res = _tc_head(x_v, params['out_w'], node0b)
    return res.reshape(B, CHECK_LEN)


# speedup vs baseline: 1.0824x; 1.0061x over previous
import functools

import jax
import jax.numpy as jnp
from jax import lax
from jax.experimental import pallas as pl
from jax.experimental.pallas import tpu as pltpu
from jax.experimental.pallas import tpu_sc as plsc

F32 = jnp.float32
HI = jax.lax.Precision.HIGHEST
BF = jnp.bfloat16


def _dot(a, b):
    return jnp.dot(a.astype(BF), b.astype(BF), preferred_element_type=F32)
NFEAT = 8
HOP = 6
NEDGE = 4
CHECK_LEN = 972
CODE_LEN = 1944
DV = 3
NTILES = 32


@functools.lru_cache(maxsize=1)
def _sc_mesh():
    return plsc.VectorSubcoreMesh(core_axis_name="c", subcore_axis_name="s")


def _tc_tables(x, wt):
    B, N, C = x.shape
    E, _, F = wt.shape
    TW = max(F, 128)

    def body(x_ref, w_ref, o_ref):
        y = _dot(x_ref[0], w_ref[0])
        if TW > F:
            y = jnp.concatenate([y, jnp.zeros((N, TW - F), F32)], axis=1)
        o_ref[0, 0] = y

    return pl.pallas_call(
        body,
        grid=(B, E),
        in_specs=[pl.BlockSpec((1, N, C), lambda b, e: (b, 0, 0)),
                  pl.BlockSpec((1, C, F), lambda b, e: (e, 0, 0))],
        out_specs=pl.BlockSpec((1, 1, N, TW), lambda b, e: (b, e, 0, 0)),
        out_shape=jax.ShapeDtypeStruct((B, E, N, TW), F32),
    )(x, wt)


def _tc_emodel(x, w1t, b1, w2t, b2):
    B, M, Cin = x.shape
    H = w1t.shape[1]
    E = w2t.shape[1]

    def body(x_ref, w1_ref, b1_ref, w2_ref, b2_ref, o_ref):
        h = jnp.maximum(
            _dot(x_ref[0], w1_ref[...])
            + b1_ref[...], 0.0)
        o_ref[0] = _dot(h, w2_ref[...]) + b2_ref[...]

    return pl.pallas_call(
        body,
        grid=(B,),
        in_specs=[pl.BlockSpec((1, M, Cin), lambda b: (b, 0, 0)),
                  pl.BlockSpec((Cin, H), lambda b: (0, 0)),
                  pl.BlockSpec((1, H), lambda b: (0, 0)),
                  pl.BlockSpec((H, E), lambda b: (0, 0)),
                  pl.BlockSpec((1, E), lambda b: (0, 0))],
        out_specs=pl.BlockSpec((1, M, E), lambda b: (b, 0, 0)),
        out_shape=jax.ShapeDtypeStruct((B, M, E), F32),
    )(x, w1t, b1, w2t, b2)


def _tc_dense(x_v, x_f0, x_f1, vf0t, bf0, wv1t, vf1t, bf1, wf1t, uvt, bv):
    B, N, C = x_v.shape
    Nf = x_f0.shape[1]
    Cf = x_f0.shape[2]
    R1 = x_f1.shape[1]
    C1 = x_f1.shape[2]
    F = bv.shape[1]

    def body(xv_ref, xf0_ref, xf1_ref, vf0_ref, bf0_ref, wv1_ref, vf1_ref,
             bf1_ref, wf1_ref, uv_ref, bv_ref, s0_ref, addv_ref, f1_ref):
        xv = xv_ref[0]
        g1 = _dot(xv, wv1_ref[...])
        m1 = jnp.max(g1, axis=0, keepdims=True)
        s1 = jnp.mean(_dot(xf1_ref[0], vf1_ref[...]), axis=0, keepdims=True)
        f1 = jnp.maximum(m1 + s1 + bf1_ref[...], 0.0)
        f1_ref[0] = f1
        t1 = _dot(f1, wf1_ref[...])
        s0_ref[0] = (_dot(xf0_ref[0], vf0_ref[...])
                     + bf0_ref[...])
        addv_ref[0] = (_dot(xv, uv_ref[...])
                       + t1 + bv_ref[...])

    return pl.pallas_call(
        body,
        grid=(B,),
        in_specs=[pl.BlockSpec((1, N, C), lambda b: (b, 0, 0)),
                  pl.BlockSpec((1, Nf, Cf), lambda b: (b, 0, 0)),
                  pl.BlockSpec((1, R1, C1), lambda b: (b, 0, 0)),
                  pl.BlockSpec((Cf, F), lambda b: (0, 0)),
                  pl.BlockSpec((1, F), lambda b: (0, 0)),
                  pl.BlockSpec((C, F), lambda b: (0, 0)),
                  pl.BlockSpec((C1, F), lambda b: (0, 0)),
                  pl.BlockSpec((1, F), lambda b: (0, 0)),
                  pl.BlockSpec((F, F), lambda b: (0, 0)),
                  pl.BlockSpec((C, F), lambda b: (0, 0)),
                  pl.BlockSpec((1, F), lambda b: (0, 0))],
        out_specs=[pl.BlockSpec((1, Nf, F), lambda b: (b, 0, 0)),
                   pl.BlockSpec((1, N, F), lambda b: (b, 0, 0)),
                   pl.BlockSpec((1, 1, F), lambda b: (b, 0, 0))],
        out_shape=[jax.ShapeDtypeStruct((B, Nf, F), F32),
                   jax.ShapeDtypeStruct((B, N, F), F32),
                   jax.ShapeDtypeStruct((B, 1, F), F32)],
    )(x_v, x_f0, x_f1, vf0t, bf0, wv1t, vf1t, bf1, wf1t, uvt, bv)


def _tc_head(x_v, wt, node0b):
    B, N, F = x_v.shape

    def body(x_ref, w_ref, n_ref, o_ref):
        xs = x_ref[0, :CHECK_LEN, :]
        r = lax.dot_general(w_ref[...].astype(BF), xs.astype(BF),
                            (((1,), (1,)), ((), ())),
                            preferred_element_type=F32)
        o_ref[0] = r + n_ref[0]

    return pl.pallas_call(
        body,
        grid=(B,),
        in_specs=[pl.BlockSpec((1, N, F), lambda b: (b, 0, 0)),
                  pl.BlockSpec((1, F), lambda b: (0, 0)),
                  pl.BlockSpec((1, 1, CHECK_LEN), lambda b: (b, 0, 0))],
        out_specs=pl.BlockSpec((1, 1, CHECK_LEN), lambda b: (b, 0, 0)),
        out_shape=jax.ShapeDtypeStruct((B, 1, CHECK_LEN), F32),
    )(x_v, wt, node0b)


def _sc_agg(table, idxm, etm, bias, F, K, NITER, SUB, PT):
    E = NEDGE
    RPD = SUB * K * E
    NTOT = NTILES * PT
    TW = max(F, 128)

    @functools.partial(
        pl.kernel,
        mesh=_sc_mesh(),
        out_type=jax.ShapeDtypeStruct((NTOT, F), F32),
        scratch_types=[
            pltpu.VMEM((NITER, RPD), jnp.int32),
            pltpu.VMEM((NITER * RPD + 16,), F32),
            pltpu.VMEM((PT, F), F32),
            pltpu.VMEM((PT, F), F32),
            pltpu.VMEM((RPD, TW), F32),
            pltpu.VMEM((RPD, TW), F32),
            pltpu.SemaphoreType.DMA,
            pltpu.SemaphoreType.DMA,
        ],
    )
    def k(table_h, idx_h, et_h, bias_h, out_h, idx_v, et_v, bias_v, out_v,
          rows_a, rows_b, sem_a, sem_b):
        wid = lax.axis_index("s") * 2 + lax.axis_index("c")
        pltpu.sync_copy(idx_h.at[wid], idx_v)
        pltpu.sync_copy(et_h.at[wid], et_v)
        pltpu.sync_copy(bias_h.at[pl.ds(wid * PT, PT)], bias_v)

        ke = K * E
        bufs = ((rows_a, sem_a), (rows_b, sem_b))

        def compute(i, rows_v):
            for d in range(SUB):
                base = d * ke
                off = i * RPD + base
                ch0 = et_v[pl.ds(off, 16)]
                ch1 = et_v[pl.ds(off + 8, 16)] if ke > 16 else None
                ets = [ch0[m] if m < 16 else ch1[m - 8] for m in range(ke)]
                dd = i * SUB + d
                for j in range(F // 16):
                    sl = pl.ds(j * 16, 16)
                    acc = None
                    for kk in range(K):
                        t = None
                        for e in range(E):
                            r = rows_v[base + kk * E + e, sl]
                            w = ets[kk * E + e]
                            t = r * w if t is None else t + r * w
                        acc = t if acc is None else jnp.maximum(acc, t)
                    out_v[dd, sl] = jnp.maximum(acc + bias_v[dd, sl], 0.0)

        pltpu.async_copy(table_h.at[idx_v.at[0]], rows_a, sem_a)
        pltpu.async_copy(table_h.at[idx_v.at[1]], rows_b, sem_b)

        def body(i2, carry):
            for half, (rows_v, sem) in enumerate(bufs):
                i = i2 * 2 + half
                pltpu.make_async_copy(table_h.at[idx_v.at[i]], rows_v, sem).wait()
                compute(i, rows_v)
                nxt = jnp.minimum(i + 2, NITER - 1)
                pltpu.async_copy(table_h.at[idx_v.at[nxt]], rows_v, sem)
            return carry

        lax.fori_loop(0, NITER // 2, body, 0)
        for rows_v, sem in bufs:
            pltpu.make_async_copy(table_h.at[idx_v.at[NITER - 1]], rows_v,
                                  sem).wait()
        pltpu.sync_copy(out_v, out_h.at[pl.ds(wid * PT, PT)])

    return k(table, idxm, etm, bias)


def _pad_rows(x, n):
    return jnp.pad(x, ((0, n - x.shape[0]), (0, 0)))


def _edge_plan(ndst_total, k):
    sub = max(1, 48 // (k * NEDGE))
    pt = -(-ndst_total // (NTILES * sub)) * sub
    pt = -(-pt // 8) * 8
    return sub, pt, pt // sub


def kernel(node_feature, hop_feature, nn_idx_f2v, nn_idx_v2f, efeature_f2v,
           efeature_v2f, params):
    B = node_feature.shape[0]

    nf0 = node_feature[..., 0]
    x_v = jnp.transpose(nf0, (0, 2, 1))
    x_f0 = jnp.transpose(hop_feature[..., 0], (0, 2, 1))
    nhop = node_feature.reshape(B, CODE_LEN, NFEAT)
    x_f1 = jnp.transpose(nhop, (0, 2, 1))

    ef = jnp.transpose(efeature_f2v, (0, 2, 3, 1)).reshape(B, CODE_LEN * DV, HOP + 1)
    ev = jnp.transpose(efeature_v2f, (0, 2, 3, 1)).reshape(B, CHECK_LEN * HOP, HOP + 1)
    pf, pv = params['emodel_f2v'], params['emodel_v2f']
    et_f2v = _tc_emodel(ef, pf['w1'].T, pf['b1'].reshape(1, -1),
                        pf['w2'].T, pf['b2'].reshape(1, -1))
    et_v2f = _tc_emodel(ev, pv['w1'].T, pv['b1'].reshape(1, -1),
                        pv['w2'].T, pv['b2'].reshape(1, -1))

    sub_v, pt_v, ni_v = _edge_plan(B * CHECK_LEN, HOP)
    sub_f, pt_f, ni_f = _edge_plan(B * CODE_LEN, DV)
    rpd_v = sub_v * HOP * NEDGE
    rpd_f = sub_f * DV * NEDGE

    b_ar = jnp.arange(B, dtype=jnp.int32)[:, None, None, None]
    e_ar = jnp.arange(NEDGE, dtype=jnp.int32)[None, None, None, :]
    rows_v2f = ((b_ar * NEDGE + e_ar) * CODE_LEN
                + nn_idx_v2f.astype(jnp.int32)[..., None])
    idxm_v2f = _pad_rows(rows_v2f.reshape(B * CHECK_LEN, HOP * NEDGE),
                         NTILES * pt_v).reshape(NTILES, ni_v, rpd_v)
    rows_f2v = ((b_ar * NEDGE + e_ar) * CHECK_LEN
                + nn_idx_f2v.astype(jnp.int32)[..., None])
    idxm_f2v = _pad_rows(rows_f2v.reshape(B * CODE_LEN, DV * NEDGE),
                         NTILES * pt_f).reshape(NTILES, ni_f, rpd_f)
    etm_v2f = jnp.pad(
        _pad_rows(et_v2f.reshape(B * CHECK_LEN, HOP * NEDGE),
                  NTILES * pt_v).reshape(NTILES, ni_v * rpd_v),
        ((0, 0), (0, 16)))
    etm_f2v = jnp.pad(
        _pad_rows(et_f2v.reshape(B * CODE_LEN, DV * NEDGE),
                  NTILES * pt_f).reshape(NTILES, ni_f * rpd_f),
        ((0, 0), (0, 16)))

    for L in params['layers']:
        F = L['bv'].shape[0]
        s0b, addv, f1 = _tc_dense(
            x_v, x_f0, x_f1,
            L['Vf0'].T, L['bf0'].reshape(1, F),
            L['Wv2f1'][0].T, L['Vf1'].T, L['bf1'].reshape(1, F),
            L['Wf2v1'][0].T, L['Uv'].T, L['bv'].reshape(1, F))

        hv = _tc_tables(x_v, jnp.transpose(L['Wv2f0'], (0, 2, 1)))
        table_v = hv.reshape(B * NEDGE * CODE_LEN, hv.shape[-1])
        bias_v = _pad_rows(s0b.reshape(B * CHECK_LEN, F), NTILES * pt_v)
        f0_flat = _sc_agg(table_v, idxm_v2f, etm_v2f, bias_v,
                          F, HOP, ni_v, sub_v, pt_v)
        f0 = f0_flat[:B * CHECK_LEN].reshape(B, CHECK_LEN, F)

        hf = _tc_tables(f0, jnp.transpose(L['Wf2v0'], (0, 2, 1)))
        table_f = hf.reshape(B * NEDGE * CHECK_LEN, hf.shape[-1])
        bias_f = _pad_rows(addv.reshape(B * CODE_LEN, F), NTILES * pt_f)
        xv_flat = _sc_agg(table_f, idxm_f2v, etm_f2v, bias_f,
                          F, DV, ni_f, sub_f, pt_f)
        x_v = xv_flat[:B * CODE_LEN].reshape(B, CODE_LEN, F)
        x_f0, x_f1 = f0, f1

    node0b = (nf0[:, :1, :CHECK_LEN] + params['out_b'][0])
    res = _tc_head(x_v, params['out_w'], node0b)
    return res.reshape(B, CHECK_LEN)


# speedup vs baseline: 2.1116x; 1.9508x over previous
import functools

import jax
import jax.numpy as jnp
from jax import lax
from jax.experimental import pallas as pl
from jax.experimental.pallas import tpu as pltpu
from jax.experimental.pallas import tpu_sc as plsc

F32 = jnp.float32
HI = jax.lax.Precision.HIGHEST
BF = jnp.bfloat16


def _dot(a, b):
    return jnp.dot(a.astype(BF), b.astype(BF), preferred_element_type=F32)
NFEAT = 8
HOP = 6
NEDGE = 4
CHECK_LEN = 972
CODE_LEN = 1944
DV = 3
NTILES = 32


@functools.lru_cache(maxsize=1)
def _sc_mesh():
    return plsc.VectorSubcoreMesh(core_axis_name="c", subcore_axis_name="s")


def _tc_tables(x, wt):
    B, N, C = x.shape
    E, _, F = wt.shape

    def body(x_ref, w_ref, o_ref):
        for e in range(E):
            o_ref[0, :, e, :] = _dot(x_ref[0], w_ref[e])

    return pl.pallas_call(
        body,
        grid=(B,),
        in_specs=[pl.BlockSpec((1, N, C), lambda b: (b, 0, 0)),
                  pl.BlockSpec((E, C, F), lambda b: (0, 0, 0))],
        out_specs=pl.BlockSpec((1, N, E, F), lambda b: (b, 0, 0, 0)),
        out_shape=jax.ShapeDtypeStruct((B, N, E, F), F32),
    )(x, wt)


def _tc_emodel(x, w1t, b1, w2t, b2):
    B, M, Cin = x.shape
    H = w1t.shape[1]
    E = w2t.shape[1]

    def body(x_ref, w1_ref, b1_ref, w2_ref, b2_ref, o_ref):
        h = jnp.maximum(
            _dot(x_ref[0], w1_ref[...])
            + b1_ref[...], 0.0)
        o_ref[0] = _dot(h, w2_ref[...]) + b2_ref[...]

    return pl.pallas_call(
        body,
        grid=(B,),
        in_specs=[pl.BlockSpec((1, M, Cin), lambda b: (b, 0, 0)),
                  pl.BlockSpec((Cin, H), lambda b: (0, 0)),
                  pl.BlockSpec((1, H), lambda b: (0, 0)),
                  pl.BlockSpec((H, E), lambda b: (0, 0)),
                  pl.BlockSpec((1, E), lambda b: (0, 0))],
        out_specs=pl.BlockSpec((1, M, E), lambda b: (b, 0, 0)),
        out_shape=jax.ShapeDtypeStruct((B, M, E), F32),
    )(x, w1t, b1, w2t, b2)


def _tc_dense(x_v, x_f0, x_f1, vf0t, bf0, wv1t, vf1t, bf1, wf1t, uvt, bv):
    B, N, C = x_v.shape
    Nf = x_f0.shape[1]
    Cf = x_f0.shape[2]
    R1 = x_f1.shape[1]
    C1 = x_f1.shape[2]
    F = bv.shape[1]

    def body(xv_ref, xf0_ref, xf1_ref, vf0_ref, bf0_ref, wv1_ref, vf1_ref,
             bf1_ref, wf1_ref, uv_ref, bv_ref, s0_ref, addv_ref, f1_ref):
        xv = xv_ref[0]
        g1 = _dot(xv, wv1_ref[...])
        m1 = jnp.max(g1, axis=0, keepdims=True)
        s1 = jnp.mean(_dot(xf1_ref[0], vf1_ref[...]), axis=0, keepdims=True)
        f1 = jnp.maximum(m1 + s1 + bf1_ref[...], 0.0)
        f1_ref[0] = f1
        t1 = _dot(f1, wf1_ref[...])
        s0_ref[0] = (_dot(xf0_ref[0], vf0_ref[...])
                     + bf0_ref[...])
        addv_ref[0] = (_dot(xv, uv_ref[...])
                       + t1 + bv_ref[...])

    return pl.pallas_call(
        body,
        grid=(B,),
        in_specs=[pl.BlockSpec((1, N, C), lambda b: (b, 0, 0)),
                  pl.BlockSpec((1, Nf, Cf), lambda b: (b, 0, 0)),
                  pl.BlockSpec((1, R1, C1), lambda b: (b, 0, 0)),
                  pl.BlockSpec((Cf, F), lambda b: (0, 0)),
                  pl.BlockSpec((1, F), lambda b: (0, 0)),
                  pl.BlockSpec((C, F), lambda b: (0, 0)),
                  pl.BlockSpec((C1, F), lambda b: (0, 0)),
                  pl.BlockSpec((1, F), lambda b: (0, 0)),
                  pl.BlockSpec((F, F), lambda b: (0, 0)),
                  pl.BlockSpec((C, F), lambda b: (0, 0)),
                  pl.BlockSpec((1, F), lambda b: (0, 0))],
        out_specs=[pl.BlockSpec((1, Nf, F), lambda b: (b, 0, 0)),
                   pl.BlockSpec((1, N, F), lambda b: (b, 0, 0)),
                   pl.BlockSpec((1, 1, F), lambda b: (b, 0, 0))],
        out_shape=[jax.ShapeDtypeStruct((B, Nf, F), F32),
                   jax.ShapeDtypeStruct((B, N, F), F32),
                   jax.ShapeDtypeStruct((B, 1, F), F32)],
    )(x_v, x_f0, x_f1, vf0t, bf0, wv1t, vf1t, bf1, wf1t, uvt, bv)


def _tc_head(x_v, wt, node0b):
    B, N, F = x_v.shape

    def body(x_ref, w_ref, n_ref, o_ref):
        xs = x_ref[0, :CHECK_LEN, :]
        r = lax.dot_general(w_ref[...].astype(BF), xs.astype(BF),
                            (((1,), (1,)), ((), ())),
                            preferred_element_type=F32)
        o_ref[0] = r + n_ref[0]

    return pl.pallas_call(
        body,
        grid=(B,),
        in_specs=[pl.BlockSpec((1, N, F), lambda b: (b, 0, 0)),
                  pl.BlockSpec((1, F), lambda b: (0, 0)),
                  pl.BlockSpec((1, 1, CHECK_LEN), lambda b: (b, 0, 0))],
        out_specs=pl.BlockSpec((1, 1, CHECK_LEN), lambda b: (b, 0, 0)),
        out_shape=jax.ShapeDtypeStruct((B, 1, CHECK_LEN), F32),
    )(x_v, wt, node0b)


def _sc_agg(table, idxm, etm, bias, F, K, NITER, SUB, PT):
    E = NEDGE
    RPD = SUB * K
    KE = K * E
    NTOT = NTILES * PT

    @functools.partial(
        pl.kernel,
        mesh=_sc_mesh(),
        out_type=jax.ShapeDtypeStruct((NTOT, F), F32),
        scratch_types=[
            pltpu.VMEM((NITER, RPD), jnp.int32),
            pltpu.VMEM((NITER * SUB * KE + 16,), F32),
            pltpu.VMEM((PT, F), F32),
            pltpu.VMEM((PT, F), F32),
            pltpu.VMEM((RPD, E * F), F32),
            pltpu.VMEM((RPD, E * F), F32),
            pltpu.SemaphoreType.DMA,
            pltpu.SemaphoreType.DMA,
        ],
    )
    def k(table_h, idx_h, et_h, bias_h, out_h, idx_v, et_v, bias_v, out_v,
          rows_a, rows_b, sem_a, sem_b):
        wid = lax.axis_index("s") * 2 + lax.axis_index("c")
        pltpu.sync_copy(idx_h.at[wid], idx_v)
        pltpu.sync_copy(et_h.at[wid], et_v)
        pltpu.sync_copy(bias_h.at[pl.ds(wid * PT, PT)], bias_v)

        bufs = ((rows_a, sem_a), (rows_b, sem_b))

        def compute(i, rows_v):
            for d in range(SUB):
                off = i * SUB * KE + d * KE
                ch0 = et_v[pl.ds(off, 16)]
                ch1 = et_v[pl.ds(off + 8, 16)] if KE > 16 else None
                ets = [ch0[m] if m < 16 else ch1[m - 8] for m in range(KE)]
                dd = i * SUB + d
                for j in range(F // 16):
                    sl = pl.ds(j * 16, 16)
                    acc = None
                    for kk in range(K):
                        t = None
                        for e in range(E):
                            r = rows_v[d * K + kk, pl.ds(e * F + j * 16, 16)]
                            w = ets[kk * E + e]
                            t = r * w if t is None else t + r * w
                        acc = t if acc is None else jnp.maximum(acc, t)
                    out_v[dd, sl] = jnp.maximum(acc + bias_v[dd, sl], 0.0)

        pltpu.async_copy(table_h.at[idx_v.at[0]], rows_a, sem_a)
        pltpu.async_copy(table_h.at[idx_v.at[1]], rows_b, sem_b)

        def body(i2, carry):
            for half, (rows_v, sem) in enumerate(bufs):
                i = i2 * 2 + half
                pltpu.make_async_copy(table_h.at[idx_v.at[i]], rows_v, sem).wait()
                compute(i, rows_v)
                nxt = jnp.minimum(i + 2, NITER - 1)
                pltpu.async_copy(table_h.at[idx_v.at[nxt]], rows_v, sem)
            return carry

        lax.fori_loop(0, NITER // 2, body, 0)
        for rows_v, sem in bufs:
            pltpu.make_async_copy(table_h.at[idx_v.at[NITER - 1]], rows_v,
                                  sem).wait()
        pltpu.sync_copy(out_v, out_h.at[pl.ds(wid * PT, PT)])

    return k(table, idxm, etm, bias)


def _pad_rows(x, n):
    return jnp.pad(x, ((0, n - x.shape[0]), (0, 0)))


def _edge_plan(ndst_total, k):
    sub = max(1, 48 // (k * NEDGE))
    pt = -(-ndst_total // (NTILES * sub)) * sub
    pt = -(-pt // 8) * 8
    return sub, pt, pt // sub


def kernel(node_feature, hop_feature, nn_idx_f2v, nn_idx_v2f, efeature_f2v,
           efeature_v2f, params):
    B = node_feature.shape[0]

    nf0 = node_feature[..., 0]
    x_v = jnp.transpose(nf0, (0, 2, 1))
    x_f0 = jnp.transpose(hop_feature[..., 0], (0, 2, 1))
    nhop = node_feature.reshape(B, CODE_LEN, NFEAT)
    x_f1 = jnp.transpose(nhop, (0, 2, 1))

    ef = jnp.transpose(efeature_f2v, (0, 2, 3, 1)).reshape(B, CODE_LEN * DV, HOP + 1)
    ev = jnp.transpose(efeature_v2f, (0, 2, 3, 1)).reshape(B, CHECK_LEN * HOP, HOP + 1)
    pf, pv = params['emodel_f2v'], params['emodel_v2f']
    et_f2v = _tc_emodel(ef, pf['w1'].T, pf['b1'].reshape(1, -1),
                        pf['w2'].T, pf['b2'].reshape(1, -1))
    et_v2f = _tc_emodel(ev, pv['w1'].T, pv['b1'].reshape(1, -1),
                        pv['w2'].T, pv['b2'].reshape(1, -1))

    sub_v, pt_v, ni_v = _edge_plan(B * CHECK_LEN, HOP)
    sub_f, pt_f, ni_f = _edge_plan(B * CODE_LEN, DV)

    b_ar = jnp.arange(B, dtype=jnp.int32)[:, None, None]
    rows_v2f = b_ar * CODE_LEN + nn_idx_v2f.astype(jnp.int32)
    idxm_v2f = _pad_rows(rows_v2f.reshape(B * CHECK_LEN, HOP),
                         NTILES * pt_v).reshape(NTILES, ni_v, sub_v * HOP)
    rows_f2v = b_ar * CHECK_LEN + nn_idx_f2v.astype(jnp.int32)
    idxm_f2v = _pad_rows(rows_f2v.reshape(B * CODE_LEN, DV),
                         NTILES * pt_f).reshape(NTILES, ni_f, sub_f * DV)
    etm_v2f = jnp.pad(
        _pad_rows(et_v2f.reshape(B * CHECK_LEN, HOP * NEDGE),
                  NTILES * pt_v).reshape(NTILES, ni_v * sub_v * HOP * NEDGE),
        ((0, 0), (0, 16)))
    etm_f2v = jnp.pad(
        _pad_rows(et_f2v.reshape(B * CODE_LEN, DV * NEDGE),
                  NTILES * pt_f).reshape(NTILES, ni_f * sub_f * DV * NEDGE),
        ((0, 0), (0, 16)))

    for L in params['layers']:
        F = L['bv'].shape[0]
        s0b, addv, f1 = _tc_dense(
            x_v, x_f0, x_f1,
            L['Vf0'].T, L['bf0'].reshape(1, F),
            L['Wv2f1'][0].T, L['Vf1'].T, L['bf1'].reshape(1, F),
            L['Wf2v1'][0].T, L['Uv'].T, L['bv'].reshape(1, F))

        hv = _tc_tables(x_v, jnp.transpose(L['Wv2f0'], (0, 2, 1)))
        table_v = hv.reshape(B * CODE_LEN, NEDGE * F)
        bias_v = _pad_rows(s0b.reshape(B * CHECK_LEN, F), NTILES * pt_v)
        f0_flat = _sc_agg(table_v, idxm_v2f, etm_v2f, bias_v,
                          F, HOP, ni_v, sub_v, pt_v)
        f0 = f0_flat[:B * CHECK_LEN].reshape(B, CHECK_LEN, F)

        hf = _tc_tables(f0, jnp.transpose(L['Wf2v0'], (0, 2, 1)))
        table_f = hf.reshape(B * CHECK_LEN, NEDGE * F)
        bias_f = _pad_rows(addv.reshape(B * CODE_LEN, F), NTILES * pt_f)
        xv_flat = _sc_agg(table_f, idxm_f2v, etm_f2v, bias_f,
                          F, DV, ni_f, sub_f, pt_f)
        x_v = xv_flat[:B * CODE_LEN].reshape(B, CODE_LEN, F)
        x_f0, x_f1 = f0, f1

    node0b = (nf0[:, :1, :CHECK_LEN] + params['out_b'][0])
    res = _tc_head(x_v, params['out_w'], node0b)
    return res.reshape(B, CHECK_LEN)


# speedup vs baseline: 2.1593x; 1.0226x over previous
import functools

import jax
import jax.numpy as jnp
from jax import lax
from jax.experimental import pallas as pl
from jax.experimental.pallas import tpu as pltpu
from jax.experimental.pallas import tpu_sc as plsc

F32 = jnp.float32
HI = jax.lax.Precision.HIGHEST
BF = jnp.bfloat16


def _dot(a, b):
    return jnp.dot(a.astype(BF), b.astype(BF), preferred_element_type=F32)
NFEAT = 8
HOP = 6
NEDGE = 4
CHECK_LEN = 972
CODE_LEN = 1944
DV = 3
NTILES = 32


@functools.lru_cache(maxsize=1)
def _sc_mesh():
    return plsc.VectorSubcoreMesh(core_axis_name="c", subcore_axis_name="s")


def _tc_tables(x, wt):
    B, N, C = x.shape
    E, _, F = wt.shape

    def body(x_ref, w_ref, o_ref):
        for e in range(E):
            o_ref[0, :, e, :] = _dot(x_ref[0], w_ref[e])

    return pl.pallas_call(
        body,
        grid=(B,),
        in_specs=[pl.BlockSpec((1, N, C), lambda b: (b, 0, 0)),
                  pl.BlockSpec((E, C, F), lambda b: (0, 0, 0))],
        out_specs=pl.BlockSpec((1, N, E, F), lambda b: (b, 0, 0, 0)),
        out_shape=jax.ShapeDtypeStruct((B, N, E, F), F32),
    )(x, wt)


def _tc_emodel(x, w1t, b1, w2t, b2):
    B, M, Cin = x.shape
    H = w1t.shape[1]
    E = w2t.shape[1]

    def body(x_ref, w1_ref, b1_ref, w2_ref, b2_ref, o_ref):
        h = jnp.maximum(
            _dot(x_ref[0], w1_ref[...])
            + b1_ref[...], 0.0)
        o_ref[0] = _dot(h, w2_ref[...]) + b2_ref[...]

    return pl.pallas_call(
        body,
        grid=(B,),
        in_specs=[pl.BlockSpec((1, M, Cin), lambda b: (b, 0, 0)),
                  pl.BlockSpec((Cin, H), lambda b: (0, 0)),
                  pl.BlockSpec((1, H), lambda b: (0, 0)),
                  pl.BlockSpec((H, E), lambda b: (0, 0)),
                  pl.BlockSpec((1, E), lambda b: (0, 0))],
        out_specs=pl.BlockSpec((1, M, E), lambda b: (b, 0, 0)),
        out_shape=jax.ShapeDtypeStruct((B, M, E), F32),
    )(x, w1t, b1, w2t, b2)


def _tc_dense(x_v, x_f0, x_f1, vf0t, bf0, wv1t, vf1t, bf1, wf1t, uvt, bv):
    B, N, C = x_v.shape
    Nf = x_f0.shape[1]
    Cf = x_f0.shape[2]
    R1 = x_f1.shape[1]
    C1 = x_f1.shape[2]
    F = bv.shape[1]

    def body(xv_ref, xf0_ref, xf1_ref, vf0_ref, bf0_ref, wv1_ref, vf1_ref,
             bf1_ref, wf1_ref, uv_ref, bv_ref, s0_ref, addv_ref, f1_ref):
        xv = xv_ref[0]
        g1 = _dot(xv, wv1_ref[...])
        m1 = jnp.max(g1, axis=0, keepdims=True)
        s1 = jnp.mean(_dot(xf1_ref[0], vf1_ref[...]), axis=0, keepdims=True)
        f1 = jnp.maximum(m1 + s1 + bf1_ref[...], 0.0)
        f1_ref[0] = f1
        t1 = _dot(f1, wf1_ref[...])
        s0_ref[0] = (_dot(xf0_ref[0], vf0_ref[...])
                     + bf0_ref[...])
        addv_ref[0] = (_dot(xv, uv_ref[...])
                       + t1 + bv_ref[...])

    return pl.pallas_call(
        body,
        grid=(B,),
        in_specs=[pl.BlockSpec((1, N, C), lambda b: (b, 0, 0)),
                  pl.BlockSpec((1, Nf, Cf), lambda b: (b, 0, 0)),
                  pl.BlockSpec((1, R1, C1), lambda b: (b, 0, 0)),
                  pl.BlockSpec((Cf, F), lambda b: (0, 0)),
                  pl.BlockSpec((1, F), lambda b: (0, 0)),
                  pl.BlockSpec((C, F), lambda b: (0, 0)),
                  pl.BlockSpec((C1, F), lambda b: (0, 0)),
                  pl.BlockSpec((1, F), lambda b: (0, 0)),
                  pl.BlockSpec((F, F), lambda b: (0, 0)),
                  pl.BlockSpec((C, F), lambda b: (0, 0)),
                  pl.BlockSpec((1, F), lambda b: (0, 0))],
        out_specs=[pl.BlockSpec((1, Nf, F), lambda b: (b, 0, 0)),
                   pl.BlockSpec((1, N, F), lambda b: (b, 0, 0)),
                   pl.BlockSpec((1, 1, F), lambda b: (b, 0, 0))],
        out_shape=[jax.ShapeDtypeStruct((B, Nf, F), F32),
                   jax.ShapeDtypeStruct((B, N, F), F32),
                   jax.ShapeDtypeStruct((B, 1, F), F32)],
    )(x_v, x_f0, x_f1, vf0t, bf0, wv1t, vf1t, bf1, wf1t, uvt, bv)


def _tc_head(x_v, wt, node0b):
    B, N, F = x_v.shape

    def body(x_ref, w_ref, n_ref, o_ref):
        xs = x_ref[0, :CHECK_LEN, :]
        r = lax.dot_general(w_ref[...].astype(BF), xs.astype(BF),
                            (((1,), (1,)), ((), ())),
                            preferred_element_type=F32)
        o_ref[0] = r + n_ref[0]

    return pl.pallas_call(
        body,
        grid=(B,),
        in_specs=[pl.BlockSpec((1, N, F), lambda b: (b, 0, 0)),
                  pl.BlockSpec((1, F), lambda b: (0, 0)),
                  pl.BlockSpec((1, 1, CHECK_LEN), lambda b: (b, 0, 0))],
        out_specs=pl.BlockSpec((1, 1, CHECK_LEN), lambda b: (b, 0, 0)),
        out_shape=jax.ShapeDtypeStruct((B, 1, CHECK_LEN), F32),
    )(x_v, wt, node0b)


def _sc_agg(table, idxm, etm, bias, F, K, NITER, SUB, PT):
    E = NEDGE
    RPD = SUB * K
    KE = K * E
    NTOT = NTILES * PT
    SL = (E * F) // 128

    @functools.partial(
        pl.kernel,
        mesh=_sc_mesh(),
        out_type=jax.ShapeDtypeStruct((NTOT, F), F32),
        scratch_types=[
            pltpu.VMEM((NITER, RPD), jnp.int32),
            pltpu.VMEM((NITER * SUB * KE + 16,), F32),
            pltpu.VMEM((PT, F), F32),
            pltpu.VMEM((PT, F), F32),
            pltpu.VMEM((RPD, SL, 128), F32),
            pltpu.VMEM((RPD, SL, 128), F32),
            pltpu.SemaphoreType.DMA,
            pltpu.SemaphoreType.DMA,
        ],
    )
    def k(table_h, idx_h, et_h, bias_h, out_h, idx_v, et_v, bias_v, out_v,
          rows_a, rows_b, sem_a, sem_b):
        wid = lax.axis_index("s") * 2 + lax.axis_index("c")
        pltpu.sync_copy(idx_h.at[wid], idx_v)
        pltpu.sync_copy(et_h.at[wid], et_v)
        pltpu.sync_copy(bias_h.at[pl.ds(wid * PT, PT)], bias_v)

        bufs = ((rows_a, sem_a), (rows_b, sem_b))

        def compute(i, rows_v):
            for d in range(SUB):
                off = i * SUB * KE + d * KE
                ch0 = et_v[pl.ds(off, 16)]
                ch1 = et_v[pl.ds(off + 8, 16)] if KE > 16 else None
                ets = [ch0[m] if m < 16 else ch1[m - 8] for m in range(KE)]
                dd = i * SUB + d
                for j in range(F // 16):
                    sl = pl.ds(j * 16, 16)
                    acc = None
                    for kk in range(K):
                        t = None
                        for e in range(E):
                            q = e * F + j * 16
                            r = rows_v[d * K + kk, q // 128, pl.ds(q % 128, 16)]
                            w = ets[kk * E + e]
                            t = r * w if t is None else t + r * w
                        acc = t if acc is None else jnp.maximum(acc, t)
                    out_v[dd, sl] = jnp.maximum(acc + bias_v[dd, sl], 0.0)

        pltpu.async_copy(table_h.at[idx_v.at[0]], rows_a, sem_a)
        pltpu.async_copy(table_h.at[idx_v.at[1]], rows_b, sem_b)

        def body(i2, carry):
            for half, (rows_v, sem) in enumerate(bufs):
                i = i2 * 2 + half
                pltpu.make_async_copy(table_h.at[idx_v.at[i]], rows_v, sem).wait()
                compute(i, rows_v)
                nxt = jnp.minimum(i + 2, NITER - 1)
                pltpu.async_copy(table_h.at[idx_v.at[nxt]], rows_v, sem)
            return carry

        lax.fori_loop(0, NITER // 2, body, 0)
        for rows_v, sem in bufs:
            pltpu.make_async_copy(table_h.at[idx_v.at[NITER - 1]], rows_v,
                                  sem).wait()
        pltpu.sync_copy(out_v, out_h.at[pl.ds(wid * PT, PT)])

    return k(table, idxm, etm, bias)


def _pad_rows(x, n):
    return jnp.pad(x, ((0, n - x.shape[0]), (0, 0)))


def _edge_plan(ndst_total, k):
    sub = max(1, 48 // (k * NEDGE))
    pt = -(-ndst_total // (NTILES * sub)) * sub
    pt = -(-pt // 8) * 8
    return sub, pt, pt // sub


def kernel(node_feature, hop_feature, nn_idx_f2v, nn_idx_v2f, efeature_f2v,
           efeature_v2f, params):
    B = node_feature.shape[0]

    nf0 = node_feature[..., 0]
    x_v = jnp.transpose(nf0, (0, 2, 1))
    x_f0 = jnp.transpose(hop_feature[..., 0], (0, 2, 1))
    nhop = node_feature.reshape(B, CODE_LEN, NFEAT)
    x_f1 = jnp.transpose(nhop, (0, 2, 1))

    ef = jnp.transpose(efeature_f2v, (0, 2, 3, 1)).reshape(B, CODE_LEN * DV, HOP + 1)
    ev = jnp.transpose(efeature_v2f, (0, 2, 3, 1)).reshape(B, CHECK_LEN * HOP, HOP + 1)
    pf, pv = params['emodel_f2v'], params['emodel_v2f']
    et_f2v = _tc_emodel(ef, pf['w1'].T, pf['b1'].reshape(1, -1),
                        pf['w2'].T, pf['b2'].reshape(1, -1))
    et_v2f = _tc_emodel(ev, pv['w1'].T, pv['b1'].reshape(1, -1),
                        pv['w2'].T, pv['b2'].reshape(1, -1))

    sub_v, pt_v, ni_v = _edge_plan(B * CHECK_LEN, HOP)
    sub_f, pt_f, ni_f = _edge_plan(B * CODE_LEN, DV)

    b_ar = jnp.arange(B, dtype=jnp.int32)[:, None, None]
    rows_v2f = b_ar * CODE_LEN + nn_idx_v2f.astype(jnp.int32)
    idxm_v2f = _pad_rows(rows_v2f.reshape(B * CHECK_LEN, HOP),
                         NTILES * pt_v).reshape(NTILES, ni_v, sub_v * HOP)
    rows_f2v = b_ar * CHECK_LEN + nn_idx_f2v.astype(jnp.int32)
    idxm_f2v = _pad_rows(rows_f2v.reshape(B * CODE_LEN, DV),
                         NTILES * pt_f).reshape(NTILES, ni_f, sub_f * DV)
    etm_v2f = jnp.pad(
        _pad_rows(et_v2f.reshape(B * CHECK_LEN, HOP * NEDGE),
                  NTILES * pt_v).reshape(NTILES, ni_v * sub_v * HOP * NEDGE),
        ((0, 0), (0, 16)))
    etm_f2v = jnp.pad(
        _pad_rows(et_f2v.reshape(B * CODE_LEN, DV * NEDGE),
                  NTILES * pt_f).reshape(NTILES, ni_f * sub_f * DV * NEDGE),
        ((0, 0), (0, 16)))

    for L in params['layers']:
        F = L['bv'].shape[0]
        s0b, addv, f1 = _tc_dense(
            x_v, x_f0, x_f1,
            L['Vf0'].T, L['bf0'].reshape(1, F),
            L['Wv2f1'][0].T, L['Vf1'].T, L['bf1'].reshape(1, F),
            L['Wf2v1'][0].T, L['Uv'].T, L['bv'].reshape(1, F))

        hv = _tc_tables(x_v, jnp.transpose(L['Wv2f0'], (0, 2, 1)))
        table_v = hv.reshape(B * CODE_LEN, (NEDGE * F) // 128, 128)
        bias_v = _pad_rows(s0b.reshape(B * CHECK_LEN, F), NTILES * pt_v)
        f0_flat = _sc_agg(table_v, idxm_v2f, etm_v2f, bias_v,
                          F, HOP, ni_v, sub_v, pt_v)
        f0 = f0_flat[:B * CHECK_LEN].reshape(B, CHECK_LEN, F)

        hf = _tc_tables(f0, jnp.transpose(L['Wf2v0'], (0, 2, 1)))
        table_f = hf.reshape(B * CHECK_LEN, (NEDGE * F) // 128, 128)
        bias_f = _pad_rows(addv.reshape(B * CODE_LEN, F), NTILES * pt_f)
        xv_flat = _sc_agg(table_f, idxm_f2v, etm_f2v, bias_f,
                          F, DV, ni_f, sub_f, pt_f)
        x_v = xv_flat[:B * CODE_LEN].reshape(B, CODE_LEN, F)
        x_f0, x_f1 = f0, f1

    node0b = (nf0[:, :1, :CHECK_LEN] + params['out_b'][0])
    res = _tc_head(x_v, params['out_w'], node0b)
    return res.reshape(B, CHECK_LEN)
